# baseline (speedup 1.0000x reference)
import functools

import jax
import jax.numpy as jnp
from jax import lax
from jax.experimental import pallas as pl
from jax.experimental.pallas import tpu as pltpu
from jax.experimental.pallas import tpu_sc as plsc

N = 10000
NE = 5000
M = 160000
D = 256

GCH = 128
NGCH = M // GCH
K = 256
C = M // K

_mesh = plsc.VectorSubcoreMesh(core_axis_name="c", subcore_axis_name="s")


def _sc_gather_body(table_hbm, idx_hbm, out_hbm, idx_v, rows_v, sem):
    wid = lax.axis_index("s") * 2 + lax.axis_index("c")

    def _step(t, carry):
        chunk = t * 32 + wid

        @pl.when(chunk < NGCH)
        def _():
            base = chunk * GCH
            pltpu.sync_copy(idx_hbm.at[pl.ds(base, GCH)], idx_v)
            pltpu.async_copy(table_hbm.at[idx_v], rows_v, sem).wait()
            pltpu.sync_copy(rows_v, out_hbm.at[pl.ds(base, GCH)])

        return carry

    lax.fori_loop(0, (NGCH + 31) // 32, _step, 0)


_sc_gather = functools.partial(
    pl.kernel,
    mesh=_mesh,
    out_type=jax.ShapeDtypeStruct((M, D), jnp.float32),
    scratch_types=[
        pltpu.VMEM((GCH,), jnp.int32),
        pltpu.VMEM((GCH, D), jnp.float32),
        pltpu.SemaphoreType.DMA,
    ],
)(_sc_gather_body)


def _mm_body(x_ref, w_ref, b_ref, o_ref):
    o_ref[...] = lax.dot_general(
        x_ref[...], w_ref[...], (((1,), (1,)), ((), ())),
        preferred_element_type=jnp.float32) + b_ref[...]


def _segsum_body(nrows, relu, idx_ref, rows_ref, o_ref, cnt_ref):
    r = pl.program_id(0)
    k = pl.program_id(1)

    @pl.when(k == 0)
    def _():
        o_ref[...] = jnp.zeros_like(o_ref)
        cnt_ref[...] = jnp.zeros_like(cnt_ref)

    row_ids = lax.broadcasted_iota(jnp.int32, (nrows, K), 0) + r * nrows
    mask = row_ids == idx_ref[...]
    o_ref[...] += lax.dot_general(
        mask.astype(jnp.bfloat16), rows_ref[...].astype(jnp.bfloat16),
        (((1,), (0,)), ((), ())),
        preferred_element_type=jnp.float32)
    cnt_ref[:, :1] += jnp.sum(mask.astype(jnp.float32), axis=1, keepdims=True)

    @pl.when(k == C - 1)
    def _():
        res = o_ref[...] / jnp.maximum(cnt_ref[:, :1], 1.0)
        if relu:
            res = jnp.maximum(res, 0.0)
        o_ref[...] = res


def _segment_mean(rows_g, idx, nseg, nblocks, relu):
    nrows = nseg // nblocks
    return pl.pallas_call(
        functools.partial(_segsum_body, nrows, relu),
        grid=(nblocks, C),
        in_specs=[
            pl.BlockSpec((1, K), lambda r, k: (0, k)),
            pl.BlockSpec((K, D), lambda r, k: (k, 0)),
        ],
        out_specs=pl.BlockSpec((nrows, D), lambda r, k: (r, 0)),
        out_shape=jax.ShapeDtypeStruct((nseg, D), jnp.float32),
        scratch_shapes=[pltpu.VMEM((nrows, 128), jnp.float32)],
    )(idx.reshape(1, M), rows_g)


def kernel(X, vertex_idx, hedge_idx, W, b):
    vidx = vertex_idx.astype(jnp.int32)
    hidx = hedge_idx.astype(jnp.int32)

    xt = pl.pallas_call(
        _mm_body,
        grid=(10,),
        in_specs=[
            pl.BlockSpec((N // 10, D), lambda i: (i, 0)),
            pl.BlockSpec((D, D), lambda i: (0, 0)),
            pl.BlockSpec((1, D), lambda i: (0, 0)),
        ],
        out_specs=pl.BlockSpec((N // 10, D), lambda i: (i, 0)),
        out_shape=jax.ShapeDtypeStruct((N, D), jnp.float32),
    )(X, W, b.reshape(1, D))

    xg = _sc_gather(xt, vidx)
    y = _segment_mean(xg, hidx, NE, 1, relu=False)

    yg = _sc_gather(y, hidx)
    xo = _segment_mean(yg, vidx, N, 2, relu=True)

    return (xo, y)


# speedup vs baseline: 1.2018x; 1.2018x over previous
import functools

import jax
import jax.numpy as jnp
from jax import lax
from jax.experimental import pallas as pl
from jax.experimental.pallas import tpu as pltpu
from jax.experimental.pallas import tpu_sc as plsc

N = 10000
NE = 5000
M = 160000
D = 256

GCH = 128
NGCH = M // GCH
K = 1280
C = M // K

_mesh = plsc.VectorSubcoreMesh(core_axis_name="c", subcore_axis_name="s")


def _sc_gather_body(table_hbm, idx_hbm, out_hbm, idx_v, rows_v, sem):
    wid = lax.axis_index("s") * 2 + lax.axis_index("c")

    def _step(t, carry):
        chunk = t * 32 + wid

        @pl.when(chunk < NGCH)
        def _():
            base = chunk * GCH
            pltpu.sync_copy(idx_hbm.at[pl.ds(base, GCH)], idx_v)
            pltpu.async_copy(table_hbm.at[idx_v], rows_v, sem).wait()
            pltpu.sync_copy(rows_v, out_hbm.at[pl.ds(base, GCH)])

        return carry

    lax.fori_loop(0, (NGCH + 31) // 32, _step, 0)


_sc_gather = functools.partial(
    pl.kernel,
    mesh=_mesh,
    out_type=jax.ShapeDtypeStruct((M, D), jnp.float32),
    scratch_types=[
        pltpu.VMEM((GCH,), jnp.int32),
        pltpu.VMEM((GCH, D), jnp.float32),
        pltpu.SemaphoreType.DMA,
    ],
)(_sc_gather_body)


def _mm_body(x_ref, w_ref, b_ref, o_ref):
    o_ref[...] = lax.dot_general(
        x_ref[...], w_ref[...], (((1,), (1,)), ((), ())),
        preferred_element_type=jnp.float32) + b_ref[...]


def _segsum_body(nrows, relu, idx_ref, rows_ref, o_ref, cnt_ref):
    r = pl.program_id(0)
    k = pl.program_id(1)

    @pl.when(k == 0)
    def _():
        o_ref[...] = jnp.zeros_like(o_ref)
        cnt_ref[...] = jnp.zeros_like(cnt_ref)

    row_ids = lax.broadcasted_iota(jnp.int32, (nrows, K), 0) + r * nrows
    mask = row_ids == idx_ref[...]
    o_ref[...] += lax.dot_general(
        mask.astype(jnp.bfloat16), rows_ref[...].astype(jnp.bfloat16),
        (((1,), (0,)), ((), ())),
        preferred_element_type=jnp.float32)
    cnt_ref[:, :1] += jnp.sum(mask.astype(jnp.float32), axis=1, keepdims=True)

    @pl.when(k == C - 1)
    def _():
        res = o_ref[...] / jnp.maximum(cnt_ref[:, :1], 1.0)
        if relu:
            res = jnp.maximum(res, 0.0)
        o_ref[...] = res


def _segment_mean(rows_g, idx, nseg, nblocks, relu):
    nrows = nseg // nblocks
    return pl.pallas_call(
        functools.partial(_segsum_body, nrows, relu),
        grid=(nblocks, C),
        in_specs=[
            pl.BlockSpec((1, K), lambda r, k: (0, k)),
            pl.BlockSpec((K, D), lambda r, k: (k, 0)),
        ],
        out_specs=pl.BlockSpec((nrows, D), lambda r, k: (r, 0)),
        out_shape=jax.ShapeDtypeStruct((nseg, D), jnp.float32),
        scratch_shapes=[pltpu.VMEM((nrows, 128), jnp.float32)],
    )(idx.reshape(1, M), rows_g)


def kernel(X, vertex_idx, hedge_idx, W, b):
    vidx = vertex_idx.astype(jnp.int32)
    hidx = hedge_idx.astype(jnp.int32)

    xt = pl.pallas_call(
        _mm_body,
        grid=(10,),
        in_specs=[
            pl.BlockSpec((N // 10, D), lambda i: (i, 0)),
            pl.BlockSpec((D, D), lambda i: (0, 0)),
            pl.BlockSpec((1, D), lambda i: (0, 0)),
        ],
        out_specs=pl.BlockSpec((N // 10, D), lambda i: (i, 0)),
        out_shape=jax.ShapeDtypeStruct((N, D), jnp.float32),
    )(X, W, b.reshape(1, D))

    xg = _sc_gather(xt, vidx)
    y = _segment_mean(xg, hidx, NE, 1, relu=False)

    yg = _sc_gather(y, hidx)
    xo = _segment_mean(yg, vidx, N, 2, relu=True)

    return (xo, y)


# speedup vs baseline: 1.2473x; 1.0378x over previous
import functools

import jax
import jax.numpy as jnp
from jax import lax
from jax.experimental import pallas as pl
from jax.experimental.pallas import tpu as pltpu
from jax.experimental.pallas import tpu_sc as plsc

N = 10000
NE = 5000
M = 160000
D = 256

GCH = 128
NGCH = M // GCH
K = 1280
C = M // K

_mesh = plsc.VectorSubcoreMesh(core_axis_name="c", subcore_axis_name="s")


def _sc_gather_body(table_hbm, idx_hbm, out_hbm, idx_v, rows_v, sem):
    wid = lax.axis_index("s") * 2 + lax.axis_index("c")

    def _step(t, carry):
        chunk = t * 32 + wid

        @pl.when(chunk < NGCH)
        def _():
            base = chunk * GCH
            pltpu.sync_copy(idx_hbm.at[pl.ds(base, GCH)], idx_v)
            pltpu.async_copy(table_hbm.at[idx_v], rows_v, sem).wait()
            pltpu.sync_copy(rows_v, out_hbm.at[pl.ds(base, GCH)])

        return carry

    lax.fori_loop(0, (NGCH + 31) // 32, _step, 0)


_sc_gather = functools.partial(
    pl.kernel,
    mesh=_mesh,
    out_type=jax.ShapeDtypeStruct((M, D), jnp.float32),
    scratch_types=[
        pltpu.VMEM((GCH,), jnp.int32),
        pltpu.VMEM((GCH, D), jnp.float32),
        pltpu.SemaphoreType.DMA,
    ],
)(_sc_gather_body)


def _mm_body(x_ref, w_ref, b_ref, o_ref):
    o_ref[...] = lax.dot_general(
        x_ref[...], w_ref[...], (((1,), (1,)), ((), ())),
        preferred_element_type=jnp.float32) + b_ref[...]


def _segsum_body(nrows, relu, idx_ref, rows_ref, o_ref, cnt_ref):
    r = pl.program_id(0)
    k = pl.program_id(1)

    @pl.when(k == 0)
    def _():
        o_ref[...] = jnp.zeros_like(o_ref)
        cnt_ref[...] = jnp.zeros_like(cnt_ref)

    row_ids = (lax.broadcasted_iota(jnp.int16, (nrows, K), 0)
               + (r * nrows).astype(jnp.int16))
    mask = row_ids == idx_ref[...]
    oht = jnp.where(mask, jnp.bfloat16(1), jnp.bfloat16(0))
    o_ref[...] += lax.dot_general(
        oht, rows_ref[...].astype(jnp.bfloat16),
        (((1,), (0,)), ((), ())),
        preferred_element_type=jnp.float32)
    cnt_ref[:, :1] += jnp.sum(oht, axis=1, keepdims=True).astype(jnp.float32)

    @pl.when(k == C - 1)
    def _():
        res = o_ref[...] / jnp.maximum(cnt_ref[:, :1], 1.0)
        if relu:
            res = jnp.maximum(res, 0.0)
        o_ref[...] = res


def _segment_mean(rows_g, idx, nseg, nblocks, relu):
    nrows = nseg // nblocks
    return pl.pallas_call(
        functools.partial(_segsum_body, nrows, relu),
        grid=(nblocks, C),
        in_specs=[
            pl.BlockSpec((1, K), lambda r, k: (0, k)),
            pl.BlockSpec((K, D), lambda r, k: (k, 0)),
        ],
        out_specs=pl.BlockSpec((nrows, D), lambda r, k: (r, 0)),
        out_shape=jax.ShapeDtypeStruct((nseg, D), jnp.float32),
        scratch_shapes=[pltpu.VMEM((nrows, 128), jnp.float32)],
    )(idx.astype(jnp.int16).reshape(1, M), rows_g)


def kernel(X, vertex_idx, hedge_idx, W, b):
    vidx = vertex_idx.astype(jnp.int32)
    hidx = hedge_idx.astype(jnp.int32)

    xt = pl.pallas_call(
        _mm_body,
        grid=(10,),
        in_specs=[
            pl.BlockSpec((N // 10, D), lambda i: (i, 0)),
            pl.BlockSpec((D, D), lambda i: (0, 0)),
            pl.BlockSpec((1, D), lambda i: (0, 0)),
        ],
        out_specs=pl.BlockSpec((N // 10, D), lambda i: (i, 0)),
        out_shape=jax.ShapeDtypeStruct((N, D), jnp.float32),
    )(X, W, b.reshape(1, D))

    xg = _sc_gather(xt, vidx)
    y = _segment_mean(xg, hidx, NE, 1, relu=False)

    yg = _sc_gather(y, hidx)
    xo = _segment_mean(yg, vidx, N, 2, relu=True)

    return (xo, y)
